# baseline (speedup 1.0000x reference)
import functools

import jax
import jax.numpy as jnp
from jax import lax
from jax.experimental import pallas as pl
from jax.experimental.pallas import tpu as pltpu
from jax.experimental.pallas import tpu_sc as plsc

BATCH = 16384
EMBED_DIM = 64
LANES = 16
CHUNK = 128
ROUND = 256


def _make_sc_products():
    info = plsc.get_sparse_core_info()
    nc, ns = info.num_cores, info.num_subcores
    nw = nc * ns
    per_w = BATCH // nw
    n_rounds = per_w // ROUND
    n_chunks = ROUND // CHUNK

    mesh = plsc.VectorSubcoreMesh(core_axis_name="c", subcore_axis_name="s")

    @functools.partial(
        pl.kernel,
        mesh=mesh,
        out_type=jax.ShapeDtypeStruct((BATCH, 128), jnp.float32),
        scratch_types=[
            pltpu.VMEM((ROUND,), jnp.int32),
            pltpu.VMEM((ROUND,), jnp.int32),
            pltpu.VMEM((ROUND,), jnp.int32),
            pltpu.VMEM((n_chunks, CHUNK), jnp.int32),
            pltpu.VMEM((n_chunks, CHUNK), jnp.int32),
            pltpu.VMEM((n_chunks, CHUNK), jnp.int32),
            pltpu.VMEM((ROUND, 128), jnp.float32),
            pltpu.VMEM((ROUND, 128), jnp.float32),
            pltpu.VMEM((ROUND, 128), jnp.float32),
            pltpu.SemaphoreType.DMA,
            pltpu.SemaphoreType.DMA,
        ],
    )
    def sc_products(head_hbm, rel_hbm, tail_hbm, ent_hbm, relemb_hbm,
                    out_hbm, idx_h, idx_r, idx_t, gh, gr, gt,
                    rows_h, rows_r, rows_t, sem, out_sem):
        wid = lax.axis_index("s") * nc + lax.axis_index("c")
        base = wid * per_w

        for r in range(n_rounds):
            rbase = base + r * ROUND
            pltpu.sync_copy(head_hbm.at[pl.ds(rbase, ROUND)], idx_h)
            pltpu.sync_copy(rel_hbm.at[pl.ds(rbase, ROUND)], idx_r)
            pltpu.sync_copy(tail_hbm.at[pl.ds(rbase, ROUND)], idx_t)

            for j in range(n_chunks):
                for g in range(CHUNK // LANES):
                    s = pl.ds(j * CHUNK + g * LANES, LANES)
                    d = pl.ds(g * LANES, LANES)
                    gh[j, d] = lax.shift_right_logical(idx_h[s], 1)
                    gr[j, d] = lax.shift_right_logical(idx_r[s], 1)
                    gt[j, d] = lax.shift_right_logical(idx_t[s], 1)

            copies = []
            for j in range(n_chunks):
                rsl = pl.ds(j * CHUNK, CHUNK)
                copies.append(pltpu.async_copy(
                    ent_hbm.at[gh.at[j]], rows_h.at[rsl], sem))
                copies.append(pltpu.async_copy(
                    relemb_hbm.at[gr.at[j]], rows_r.at[rsl], sem))
                copies.append(pltpu.async_copy(
                    ent_hbm.at[gt.at[j]], rows_t.at[rsl], sem))
            for c in copies:
                c.wait()

            def group(g, _):
                gsl = pl.ds(g * LANES, LANES)
                ovh = (idx_h[gsl] & 1) * EMBED_DIM
                ovr = (idx_r[gsl] & 1) * EMBED_DIM
                ovt = (idx_t[gsl] & 1) * EMBED_DIM
                for k in range(LANES):
                    i = g * LANES + k
                    oh, orr, ot = ovh[k], ovr[k], ovt[k]
                    for c in range(EMBED_DIM // LANES):
                        p = (rows_h[i, pl.ds(oh + c * LANES, LANES)]
                             * rows_r[i, pl.ds(orr + c * LANES, LANES)]
                             * rows_t[i, pl.ds(ot + c * LANES, LANES)])
                        rows_h[i, pl.ds(c * LANES, LANES)] = p
                return 0

            lax.fori_loop(0, ROUND // LANES, group, 0)

            out_cp = pltpu.async_copy(
                rows_h, out_hbm.at[pl.ds(rbase, ROUND)], out_sem)
            out_cp.wait()

    return sc_products


_sc_products = _make_sc_products()


def _tc_reduce_body(p_ref, out_ref):
    out_ref[...] = jnp.sum(p_ref[:, :EMBED_DIM], axis=1)


_TC_BLOCK = 2048


def _tc_reduce(products):
    return pl.pallas_call(
        _tc_reduce_body,
        out_shape=jax.ShapeDtypeStruct((BATCH,), jnp.float32),
        grid=(BATCH // _TC_BLOCK,),
        in_specs=[pl.BlockSpec((_TC_BLOCK, 128), lambda i: (i, 0))],
        out_specs=pl.BlockSpec((_TC_BLOCK,), lambda i: (i,)),
    )(products)


def kernel(head, relation, tail, entity_embeddings, relation_embeddings):
    ent2 = entity_embeddings.reshape(-1, 128)
    rel2 = relation_embeddings.reshape(-1, 128)
    products = _sc_products(head, relation, tail, ent2, rel2)
    return _tc_reduce(products)


# speedup vs baseline: 1.7125x; 1.7125x over previous
import functools

import jax
import jax.numpy as jnp
from jax import lax
from jax.experimental import pallas as pl
from jax.experimental.pallas import tpu as pltpu
from jax.experimental.pallas import tpu_sc as plsc

BATCH = 16384
EMBED_DIM = 64
LANES = 16
CHUNK = 128
ROUND = 256

NE = 1000000
_TW = 2048
HALFN = 245 * _TW


def _transpose_body(a_ref, b_ref, out_ref):
    out_ref[:, :EMBED_DIM] = a_ref[...].T
    out_ref[:, EMBED_DIM:] = b_ref[...].T


def _to_gatherable(ent_t):
    return pl.pallas_call(
        _transpose_body,
        out_shape=jax.ShapeDtypeStruct((HALFN, 128), jnp.float32),
        grid=(HALFN // _TW,),
        in_specs=[
            pl.BlockSpec((EMBED_DIM, _TW), lambda j: (0, j)),
            pl.BlockSpec(
                (EMBED_DIM, _TW),
                lambda j: (0, jnp.minimum(j + HALFN // _TW,
                                          (NE - 1) // _TW))),
        ],
        out_specs=pl.BlockSpec((_TW, 128), lambda j: (j, 0)),
    )(ent_t, ent_t)


def _make_sc_products():
    info = plsc.get_sparse_core_info()
    nc, ns = info.num_cores, info.num_subcores
    nw = nc * ns
    per_w = BATCH // nw
    n_rounds = per_w // ROUND
    n_chunks = ROUND // CHUNK

    mesh = plsc.VectorSubcoreMesh(core_axis_name="c", subcore_axis_name="s")

    @functools.partial(
        pl.kernel,
        mesh=mesh,
        out_type=jax.ShapeDtypeStruct((BATCH, 128), jnp.float32),
        scratch_types=[
            pltpu.VMEM((ROUND,), jnp.int32),
            pltpu.VMEM((ROUND,), jnp.int32),
            pltpu.VMEM((ROUND,), jnp.int32),
            pltpu.VMEM((n_chunks, CHUNK), jnp.int32),
            pltpu.VMEM((n_chunks, CHUNK), jnp.int32),
            pltpu.VMEM((n_chunks, CHUNK), jnp.int32),
            pltpu.VMEM((ROUND, 128), jnp.float32),
            pltpu.VMEM((ROUND, 128), jnp.float32),
            pltpu.VMEM((ROUND, 128), jnp.float32),
            pltpu.SemaphoreType.DMA,
            pltpu.SemaphoreType.DMA,
        ],
    )
    def sc_products(head_hbm, rel_hbm, tail_hbm, ent_hbm, relemb_hbm,
                    out_hbm, idx_h, idx_r, idx_t, gh, gr, gt,
                    rows_h, rows_r, rows_t, sem, out_sem):
        wid = lax.axis_index("s") * nc + lax.axis_index("c")
        base = wid * per_w

        for r in range(n_rounds):
            rbase = base + r * ROUND
            pltpu.sync_copy(head_hbm.at[pl.ds(rbase, ROUND)], idx_h)
            pltpu.sync_copy(rel_hbm.at[pl.ds(rbase, ROUND)], idx_r)
            pltpu.sync_copy(tail_hbm.at[pl.ds(rbase, ROUND)], idx_t)

            for j in range(n_chunks):
                for g in range(CHUNK // LANES):
                    s = pl.ds(j * CHUNK + g * LANES, LANES)
                    d = pl.ds(g * LANES, LANES)
                    hv = idx_h[s]
                    tv = idx_t[s]
                    gh[j, d] = hv - jnp.where(hv >= HALFN, HALFN, 0)
                    gt[j, d] = tv - jnp.where(tv >= HALFN, HALFN, 0)
                    gr[j, d] = lax.shift_right_logical(idx_r[s], 1)

            copies = []
            for j in range(n_chunks):
                rsl = pl.ds(j * CHUNK, CHUNK)
                copies.append(pltpu.async_copy(
                    ent_hbm.at[gh.at[j]], rows_h.at[rsl], sem))
                copies.append(pltpu.async_copy(
                    relemb_hbm.at[gr.at[j]], rows_r.at[rsl], sem))
                copies.append(pltpu.async_copy(
                    ent_hbm.at[gt.at[j]], rows_t.at[rsl], sem))
            for c in copies:
                c.wait()

            def group(g, _):
                gsl = pl.ds(g * LANES, LANES)
                ovh = jnp.where(idx_h[gsl] >= HALFN, EMBED_DIM, 0)
                ovt = jnp.where(idx_t[gsl] >= HALFN, EMBED_DIM, 0)
                ovr = (idx_r[gsl] & 1) * EMBED_DIM
                for k in range(LANES):
                    i = g * LANES + k
                    oh, orr, ot = ovh[k], ovr[k], ovt[k]
                    for c in range(EMBED_DIM // LANES):
                        p = (rows_h[i, pl.ds(oh + c * LANES, LANES)]
                             * rows_r[i, pl.ds(orr + c * LANES, LANES)]
                             * rows_t[i, pl.ds(ot + c * LANES, LANES)])
                        rows_h[i, pl.ds(c * LANES, LANES)] = p
                return 0

            lax.fori_loop(0, ROUND // LANES, group, 0)

            out_cp = pltpu.async_copy(
                rows_h, out_hbm.at[pl.ds(rbase, ROUND)], out_sem)
            out_cp.wait()

    return sc_products


_sc_products = _make_sc_products()


def _tc_reduce_body(p_ref, out_ref):
    out_ref[...] = jnp.sum(p_ref[:, :EMBED_DIM], axis=1)


_TC_BLOCK = 2048


def _tc_reduce(products):
    return pl.pallas_call(
        _tc_reduce_body,
        out_shape=jax.ShapeDtypeStruct((BATCH,), jnp.float32),
        grid=(BATCH // _TC_BLOCK,),
        in_specs=[pl.BlockSpec((_TC_BLOCK, 128), lambda i: (i, 0))],
        out_specs=pl.BlockSpec((_TC_BLOCK,), lambda i: (i,)),
    )(products)


def kernel(head, relation, tail, entity_embeddings, relation_embeddings):
    ent2 = _to_gatherable(entity_embeddings.T)
    rel2 = relation_embeddings.reshape(-1, 128)
    products = _sc_products(head, relation, tail, ent2, rel2)
    return _tc_reduce(products)


# speedup vs baseline: 1.7128x; 1.0002x over previous
import functools

import jax
import jax.numpy as jnp
from jax import lax
from jax.experimental import pallas as pl
from jax.experimental.pallas import tpu as pltpu
from jax.experimental.pallas import tpu_sc as plsc

BATCH = 16384
EMBED_DIM = 64
LANES = 16
CHUNK = 128
ROUND = 256

NE = 1000000
_TW = 2048
HALFN = 245 * _TW


_DIMS = (((0,), (0,)), ((), ()))


def _transpose_body(a_ref, b_ref, out_ref):
    eye = jnp.eye(EMBED_DIM, dtype=jnp.float32)
    out_ref[:, :EMBED_DIM] = lax.dot_general(
        a_ref[...], eye, _DIMS, preferred_element_type=jnp.float32)
    out_ref[:, EMBED_DIM:] = lax.dot_general(
        b_ref[...], eye, _DIMS, preferred_element_type=jnp.float32)


def _to_gatherable(ent_t):
    return pl.pallas_call(
        _transpose_body,
        out_shape=jax.ShapeDtypeStruct((HALFN, 128), jnp.float32),
        grid=(HALFN // _TW,),
        in_specs=[
            pl.BlockSpec((EMBED_DIM, _TW), lambda j: (0, j)),
            pl.BlockSpec(
                (EMBED_DIM, _TW),
                lambda j: (0, jnp.minimum(j + HALFN // _TW,
                                          (NE - 1) // _TW))),
        ],
        out_specs=pl.BlockSpec((_TW, 128), lambda j: (j, 0)),
    )(ent_t, ent_t)


def _make_sc_products():
    info = plsc.get_sparse_core_info()
    nc, ns = info.num_cores, info.num_subcores
    nw = nc * ns
    per_w = BATCH // nw
    n_rounds = per_w // ROUND
    n_chunks = ROUND // CHUNK

    mesh = plsc.VectorSubcoreMesh(core_axis_name="c", subcore_axis_name="s")

    @functools.partial(
        pl.kernel,
        mesh=mesh,
        out_type=jax.ShapeDtypeStruct((BATCH, 128), jnp.float32),
        scratch_types=[
            pltpu.VMEM((ROUND,), jnp.int32),
            pltpu.VMEM((ROUND,), jnp.int32),
            pltpu.VMEM((ROUND,), jnp.int32),
            pltpu.VMEM((n_chunks, CHUNK), jnp.int32),
            pltpu.VMEM((n_chunks, CHUNK), jnp.int32),
            pltpu.VMEM((n_chunks, CHUNK), jnp.int32),
            pltpu.VMEM((ROUND, 128), jnp.float32),
            pltpu.VMEM((ROUND, 128), jnp.float32),
            pltpu.VMEM((ROUND, 128), jnp.float32),
            pltpu.SemaphoreType.DMA,
            pltpu.SemaphoreType.DMA,
        ],
    )
    def sc_products(head_hbm, rel_hbm, tail_hbm, ent_hbm, relemb_hbm,
                    out_hbm, idx_h, idx_r, idx_t, gh, gr, gt,
                    rows_h, rows_r, rows_t, sem, out_sem):
        wid = lax.axis_index("s") * nc + lax.axis_index("c")
        base = wid * per_w

        for r in range(n_rounds):
            rbase = base + r * ROUND
            pltpu.sync_copy(head_hbm.at[pl.ds(rbase, ROUND)], idx_h)
            pltpu.sync_copy(rel_hbm.at[pl.ds(rbase, ROUND)], idx_r)
            pltpu.sync_copy(tail_hbm.at[pl.ds(rbase, ROUND)], idx_t)

            for j in range(n_chunks):
                for g in range(CHUNK // LANES):
                    s = pl.ds(j * CHUNK + g * LANES, LANES)
                    d = pl.ds(g * LANES, LANES)
                    hv = idx_h[s]
                    tv = idx_t[s]
                    gh[j, d] = hv - jnp.where(hv >= HALFN, HALFN, 0)
                    gt[j, d] = tv - jnp.where(tv >= HALFN, HALFN, 0)
                    gr[j, d] = lax.shift_right_logical(idx_r[s], 1)

            copies = []
            for j in range(n_chunks):
                rsl = pl.ds(j * CHUNK, CHUNK)
                copies.append(pltpu.async_copy(
                    ent_hbm.at[gh.at[j]], rows_h.at[rsl], sem))
                copies.append(pltpu.async_copy(
                    relemb_hbm.at[gr.at[j]], rows_r.at[rsl], sem))
                copies.append(pltpu.async_copy(
                    ent_hbm.at[gt.at[j]], rows_t.at[rsl], sem))
            for c in copies:
                c.wait()

            def group(g, _):
                gsl = pl.ds(g * LANES, LANES)
                ovh = jnp.where(idx_h[gsl] >= HALFN, EMBED_DIM, 0)
                ovt = jnp.where(idx_t[gsl] >= HALFN, EMBED_DIM, 0)
                ovr = (idx_r[gsl] & 1) * EMBED_DIM
                for k in range(LANES):
                    i = g * LANES + k
                    oh, orr, ot = ovh[k], ovr[k], ovt[k]
                    for c in range(EMBED_DIM // LANES):
                        p = (rows_h[i, pl.ds(oh + c * LANES, LANES)]
                             * rows_r[i, pl.ds(orr + c * LANES, LANES)]
                             * rows_t[i, pl.ds(ot + c * LANES, LANES)])
                        rows_h[i, pl.ds(c * LANES, LANES)] = p
                return 0

            lax.fori_loop(0, ROUND // LANES, group, 0)

            out_cp = pltpu.async_copy(
                rows_h, out_hbm.at[pl.ds(rbase, ROUND)], out_sem)
            out_cp.wait()

    return sc_products


_sc_products = _make_sc_products()


def _tc_reduce_body(p_ref, out_ref):
    out_ref[...] = jnp.sum(p_ref[:, :EMBED_DIM], axis=1)


_TC_BLOCK = 2048


def _tc_reduce(products):
    return pl.pallas_call(
        _tc_reduce_body,
        out_shape=jax.ShapeDtypeStruct((BATCH,), jnp.float32),
        grid=(BATCH // _TC_BLOCK,),
        in_specs=[pl.BlockSpec((_TC_BLOCK, 128), lambda i: (i, 0))],
        out_specs=pl.BlockSpec((_TC_BLOCK,), lambda i: (i,)),
    )(products)


def kernel(head, relation, tail, entity_embeddings, relation_embeddings):
    ent2 = _to_gatherable(entity_embeddings.T)
    rel2 = relation_embeddings.reshape(-1, 128)
    products = _sc_products(head, relation, tail, ent2, rel2)
    return _tc_reduce(products)


# speedup vs baseline: 1.8477x; 1.0788x over previous
import functools

import jax
import jax.numpy as jnp
from jax import lax
from jax.experimental import pallas as pl
from jax.experimental.pallas import tpu as pltpu
from jax.experimental.pallas import tpu_sc as plsc

BATCH = 16384
EMBED_DIM = 64
LANES = 16
CHUNK = 128
ROUND = 256

NE = 1000000
_TW = 2048
HALFN = 245 * _TW


_DIMS = (((0,), (0,)), ((), ()))


def _transpose_body(a_ref, b_ref, out_ref):
    eye = jnp.eye(EMBED_DIM, dtype=jnp.bfloat16)
    out_ref[:, :EMBED_DIM] = lax.dot_general(
        a_ref[...].astype(jnp.bfloat16), eye, _DIMS,
        preferred_element_type=jnp.float32)
    out_ref[:, EMBED_DIM:] = lax.dot_general(
        b_ref[...].astype(jnp.bfloat16), eye, _DIMS,
        preferred_element_type=jnp.float32)


def _to_gatherable(ent_t):
    return pl.pallas_call(
        _transpose_body,
        out_shape=jax.ShapeDtypeStruct((HALFN, 128), jnp.float32),
        grid=(HALFN // _TW,),
        in_specs=[
            pl.BlockSpec((EMBED_DIM, _TW), lambda j: (0, j)),
            pl.BlockSpec(
                (EMBED_DIM, _TW),
                lambda j: (0, jnp.minimum(j + HALFN // _TW,
                                          (NE - 1) // _TW))),
        ],
        out_specs=pl.BlockSpec((_TW, 128), lambda j: (j, 0)),
    )(ent_t, ent_t)


def _make_sc_products():
    info = plsc.get_sparse_core_info()
    nc, ns = info.num_cores, info.num_subcores
    nw = nc * ns
    per_w = BATCH // nw
    n_rounds = per_w // ROUND
    n_chunks = ROUND // CHUNK

    mesh = plsc.VectorSubcoreMesh(core_axis_name="c", subcore_axis_name="s")

    @functools.partial(
        pl.kernel,
        mesh=mesh,
        out_type=jax.ShapeDtypeStruct((BATCH, 128), jnp.float32),
        scratch_types=[
            pltpu.VMEM((ROUND,), jnp.int32),
            pltpu.VMEM((ROUND,), jnp.int32),
            pltpu.VMEM((ROUND,), jnp.int32),
            pltpu.VMEM((n_chunks, CHUNK), jnp.int32),
            pltpu.VMEM((n_chunks, CHUNK), jnp.int32),
            pltpu.VMEM((n_chunks, CHUNK), jnp.int32),
            pltpu.VMEM((ROUND, 128), jnp.float32),
            pltpu.VMEM((ROUND, 128), jnp.float32),
            pltpu.VMEM((ROUND, 128), jnp.float32),
            pltpu.SemaphoreType.DMA,
            pltpu.SemaphoreType.DMA,
        ],
    )
    def sc_products(head_hbm, rel_hbm, tail_hbm, ent_hbm, relemb_hbm,
                    out_hbm, idx_h, idx_r, idx_t, gh, gr, gt,
                    rows_h, rows_r, rows_t, sem, out_sem):
        wid = lax.axis_index("s") * nc + lax.axis_index("c")
        base = wid * per_w

        for r in range(n_rounds):
            rbase = base + r * ROUND
            pltpu.sync_copy(head_hbm.at[pl.ds(rbase, ROUND)], idx_h)
            pltpu.sync_copy(rel_hbm.at[pl.ds(rbase, ROUND)], idx_r)
            pltpu.sync_copy(tail_hbm.at[pl.ds(rbase, ROUND)], idx_t)

            for j in range(n_chunks):
                for g in range(CHUNK // LANES):
                    s = pl.ds(j * CHUNK + g * LANES, LANES)
                    d = pl.ds(g * LANES, LANES)
                    hv = idx_h[s]
                    tv = idx_t[s]
                    gh[j, d] = hv - jnp.where(hv >= HALFN, HALFN, 0)
                    gt[j, d] = tv - jnp.where(tv >= HALFN, HALFN, 0)
                    gr[j, d] = lax.shift_right_logical(idx_r[s], 1)

            copies = []
            for j in range(n_chunks):
                rsl = pl.ds(j * CHUNK, CHUNK)
                copies.append(pltpu.async_copy(
                    ent_hbm.at[gh.at[j]], rows_h.at[rsl], sem))
                copies.append(pltpu.async_copy(
                    relemb_hbm.at[gr.at[j]], rows_r.at[rsl], sem))
                copies.append(pltpu.async_copy(
                    ent_hbm.at[gt.at[j]], rows_t.at[rsl], sem))
            for c in copies:
                c.wait()

            def group(g, _):
                gsl = pl.ds(g * LANES, LANES)
                ovh = jnp.where(idx_h[gsl] >= HALFN, EMBED_DIM, 0)
                ovt = jnp.where(idx_t[gsl] >= HALFN, EMBED_DIM, 0)
                ovr = (idx_r[gsl] & 1) * EMBED_DIM
                for k in range(LANES):
                    i = g * LANES + k
                    oh, orr, ot = ovh[k], ovr[k], ovt[k]
                    for c in range(EMBED_DIM // LANES):
                        p = (rows_h[i, pl.ds(oh + c * LANES, LANES)]
                             * rows_r[i, pl.ds(orr + c * LANES, LANES)]
                             * rows_t[i, pl.ds(ot + c * LANES, LANES)])
                        rows_h[i, pl.ds(c * LANES, LANES)] = p
                return 0

            lax.fori_loop(0, ROUND // LANES, group, 0)

            out_cp = pltpu.async_copy(
                rows_h, out_hbm.at[pl.ds(rbase, ROUND)], out_sem)
            out_cp.wait()

    return sc_products


_sc_products = _make_sc_products()


def _tc_reduce_body(p_ref, out_ref):
    out_ref[...] = jnp.sum(p_ref[:, :EMBED_DIM], axis=1)


_TC_BLOCK = 2048


def _tc_reduce(products):
    return pl.pallas_call(
        _tc_reduce_body,
        out_shape=jax.ShapeDtypeStruct((BATCH,), jnp.float32),
        grid=(BATCH // _TC_BLOCK,),
        in_specs=[pl.BlockSpec((_TC_BLOCK, 128), lambda i: (i, 0))],
        out_specs=pl.BlockSpec((_TC_BLOCK,), lambda i: (i,)),
    )(products)


def kernel(head, relation, tail, entity_embeddings, relation_embeddings):
    ent2 = _to_gatherable(entity_embeddings.T)
    rel2 = relation_embeddings.reshape(-1, 128)
    products = _sc_products(head, relation, tail, ent2, rel2)
    return _tc_reduce(products)


# speedup vs baseline: 2.2937x; 1.2414x over previous
import functools

import jax
import jax.numpy as jnp
from jax import lax
from jax.experimental import pallas as pl
from jax.experimental.pallas import tpu as pltpu
from jax.experimental.pallas import tpu_sc as plsc

BATCH = 16384
EMBED_DIM = 64
LANES = 16
CHUNK = 128
ROUND = 256

NE = 1000000
_TW = 4096
HALFN = 123 * _TW


_DIMS = (((0,), (0,)), ((), ()))


def _transpose_body(a_ref, b_ref, out_ref):
    eye = jnp.eye(EMBED_DIM, dtype=jnp.bfloat16)
    out_ref[:, :EMBED_DIM] = lax.dot_general(
        a_ref[...].astype(jnp.bfloat16), eye, _DIMS,
        preferred_element_type=jnp.float32)
    out_ref[:, EMBED_DIM:] = lax.dot_general(
        b_ref[...].astype(jnp.bfloat16), eye, _DIMS,
        preferred_element_type=jnp.float32)


def _to_gatherable(ent_t):
    return pl.pallas_call(
        _transpose_body,
        out_shape=jax.ShapeDtypeStruct((HALFN, 128), jnp.float32),
        grid=(HALFN // _TW,),
        in_specs=[
            pl.BlockSpec((EMBED_DIM, _TW), lambda j: (0, j)),
            pl.BlockSpec(
                (EMBED_DIM, _TW),
                lambda j: (0, jnp.minimum(j + HALFN // _TW,
                                          (NE - 1) // _TW))),
        ],
        out_specs=pl.BlockSpec((_TW, 128), lambda j: (j, 0)),
    )(ent_t, ent_t)


def _make_sc_products():
    info = plsc.get_sparse_core_info()
    nc, ns = info.num_cores, info.num_subcores
    nw = nc * ns
    per_w = BATCH // nw
    n_rounds = per_w // ROUND
    n_chunks = ROUND // CHUNK

    mesh = plsc.VectorSubcoreMesh(core_axis_name="c", subcore_axis_name="s")

    @functools.partial(
        pl.kernel,
        mesh=mesh,
        out_type=jax.ShapeDtypeStruct((BATCH, 128), jnp.float32),
        scratch_types=[
            pltpu.VMEM((ROUND,), jnp.int32),
            pltpu.VMEM((ROUND,), jnp.int32),
            pltpu.VMEM((ROUND,), jnp.int32),
            pltpu.VMEM((n_chunks, CHUNK), jnp.int32),
            pltpu.VMEM((n_chunks, CHUNK), jnp.int32),
            pltpu.VMEM((n_chunks, CHUNK), jnp.int32),
            pltpu.VMEM((ROUND, 128), jnp.float32),
            pltpu.VMEM((ROUND, 128), jnp.float32),
            pltpu.VMEM((ROUND, 128), jnp.float32),
            pltpu.SemaphoreType.DMA,
            pltpu.SemaphoreType.DMA,
        ],
    )
    def sc_products(head_hbm, rel_hbm, tail_hbm, ent_hbm, relemb_hbm,
                    out_hbm, idx_h, idx_r, idx_t, gh, gr, gt,
                    rows_h, rows_r, rows_t, sem, out_sem):
        wid = lax.axis_index("s") * nc + lax.axis_index("c")
        base = wid * per_w

        for r in range(n_rounds):
            rbase = base + r * ROUND
            pltpu.sync_copy(head_hbm.at[pl.ds(rbase, ROUND)], idx_h)
            pltpu.sync_copy(rel_hbm.at[pl.ds(rbase, ROUND)], idx_r)
            pltpu.sync_copy(tail_hbm.at[pl.ds(rbase, ROUND)], idx_t)

            for j in range(n_chunks):
                for g in range(CHUNK // LANES):
                    s = pl.ds(j * CHUNK + g * LANES, LANES)
                    d = pl.ds(g * LANES, LANES)
                    hv = idx_h[s]
                    tv = idx_t[s]
                    gh[j, d] = hv - jnp.where(hv >= HALFN, HALFN, 0)
                    gt[j, d] = tv - jnp.where(tv >= HALFN, HALFN, 0)
                    gr[j, d] = lax.shift_right_logical(idx_r[s], 1)

            copies = []
            for j in range(n_chunks):
                rsl = pl.ds(j * CHUNK, CHUNK)
                copies.append(pltpu.async_copy(
                    ent_hbm.at[gh.at[j]], rows_h.at[rsl], sem))
                copies.append(pltpu.async_copy(
                    relemb_hbm.at[gr.at[j]], rows_r.at[rsl], sem))
                copies.append(pltpu.async_copy(
                    ent_hbm.at[gt.at[j]], rows_t.at[rsl], sem))
            for c in copies:
                c.wait()

            def group(g, _):
                gsl = pl.ds(g * LANES, LANES)
                ovh = jnp.where(idx_h[gsl] >= HALFN, EMBED_DIM, 0)
                ovt = jnp.where(idx_t[gsl] >= HALFN, EMBED_DIM, 0)
                ovr = (idx_r[gsl] & 1) * EMBED_DIM
                for k in range(LANES):
                    i = g * LANES + k
                    oh, orr, ot = ovh[k], ovr[k], ovt[k]
                    for c in range(EMBED_DIM // LANES):
                        p = (rows_h[i, pl.ds(oh + c * LANES, LANES)]
                             * rows_r[i, pl.ds(orr + c * LANES, LANES)]
                             * rows_t[i, pl.ds(ot + c * LANES, LANES)])
                        rows_h[i, pl.ds(c * LANES, LANES)] = p
                return 0

            lax.fori_loop(0, ROUND // LANES, group, 0)

            out_cp = pltpu.async_copy(
                rows_h, out_hbm.at[pl.ds(rbase, ROUND)], out_sem)
            out_cp.wait()

    return sc_products


_sc_products = _make_sc_products()


def _tc_reduce_body(p_ref, out_ref):
    out_ref[...] = jnp.sum(p_ref[:, :EMBED_DIM], axis=1)


_TC_BLOCK = 2048


def _tc_reduce(products):
    return pl.pallas_call(
        _tc_reduce_body,
        out_shape=jax.ShapeDtypeStruct((BATCH,), jnp.float32),
        grid=(BATCH // _TC_BLOCK,),
        in_specs=[pl.BlockSpec((_TC_BLOCK, 128), lambda i: (i, 0))],
        out_specs=pl.BlockSpec((_TC_BLOCK,), lambda i: (i,)),
    )(products)


def kernel(head, relation, tail, entity_embeddings, relation_embeddings):
    ent2 = _to_gatherable(entity_embeddings.T)
    rel2 = relation_embeddings.reshape(-1, 128)
    products = _sc_products(head, relation, tail, ent2, rel2)
    return _tc_reduce(products)


# speedup vs baseline: 2.6078x; 1.1370x over previous
import functools

import jax
import jax.numpy as jnp
from jax import lax
from jax.experimental import pallas as pl
from jax.experimental.pallas import tpu as pltpu
from jax.experimental.pallas import tpu_sc as plsc

BATCH = 16384
EMBED_DIM = 64
LANES = 16
CHUNK = 128
ROUND = 256

NE = 1000000
_TW = 8192
HALFN = 62 * _TW


_DIMS = (((0,), (0,)), ((), ()))


def _transpose_body(a_ref, b_ref, out_ref):
    eye = jnp.eye(EMBED_DIM, dtype=jnp.bfloat16)
    out_ref[:, :EMBED_DIM] = lax.dot_general(
        a_ref[...].astype(jnp.bfloat16), eye, _DIMS,
        preferred_element_type=jnp.float32)
    out_ref[:, EMBED_DIM:] = lax.dot_general(
        b_ref[...].astype(jnp.bfloat16), eye, _DIMS,
        preferred_element_type=jnp.float32)


def _to_gatherable(ent_t):
    return pl.pallas_call(
        _transpose_body,
        out_shape=jax.ShapeDtypeStruct((HALFN, 128), jnp.float32),
        grid=(HALFN // _TW,),
        in_specs=[
            pl.BlockSpec((EMBED_DIM, _TW), lambda j: (0, j)),
            pl.BlockSpec(
                (EMBED_DIM, _TW),
                lambda j: (0, jnp.minimum(j + HALFN // _TW,
                                          (NE - 1) // _TW))),
        ],
        out_specs=pl.BlockSpec((_TW, 128), lambda j: (j, 0)),
    )(ent_t, ent_t)


def _make_sc_products():
    info = plsc.get_sparse_core_info()
    nc, ns = info.num_cores, info.num_subcores
    nw = nc * ns
    per_w = BATCH // nw
    n_rounds = per_w // ROUND
    n_chunks = ROUND // CHUNK

    mesh = plsc.VectorSubcoreMesh(core_axis_name="c", subcore_axis_name="s")

    @functools.partial(
        pl.kernel,
        mesh=mesh,
        out_type=jax.ShapeDtypeStruct((BATCH, 128), jnp.float32),
        scratch_types=[
            pltpu.VMEM((ROUND,), jnp.int32),
            pltpu.VMEM((ROUND,), jnp.int32),
            pltpu.VMEM((ROUND,), jnp.int32),
            pltpu.VMEM((n_chunks, CHUNK), jnp.int32),
            pltpu.VMEM((n_chunks, CHUNK), jnp.int32),
            pltpu.VMEM((n_chunks, CHUNK), jnp.int32),
            pltpu.VMEM((ROUND, 128), jnp.float32),
            pltpu.VMEM((ROUND, 128), jnp.float32),
            pltpu.VMEM((ROUND, 128), jnp.float32),
            pltpu.SemaphoreType.DMA,
            pltpu.SemaphoreType.DMA,
        ],
    )
    def sc_products(head_hbm, rel_hbm, tail_hbm, ent_hbm, relemb_hbm,
                    out_hbm, idx_h, idx_r, idx_t, gh, gr, gt,
                    rows_h, rows_r, rows_t, sem, out_sem):
        wid = lax.axis_index("s") * nc + lax.axis_index("c")
        base = wid * per_w

        for r in range(n_rounds):
            rbase = base + r * ROUND
            pltpu.sync_copy(head_hbm.at[pl.ds(rbase, ROUND)], idx_h)
            pltpu.sync_copy(rel_hbm.at[pl.ds(rbase, ROUND)], idx_r)
            pltpu.sync_copy(tail_hbm.at[pl.ds(rbase, ROUND)], idx_t)

            for j in range(n_chunks):
                for g in range(CHUNK // LANES):
                    s = pl.ds(j * CHUNK + g * LANES, LANES)
                    d = pl.ds(g * LANES, LANES)
                    hv = idx_h[s]
                    tv = idx_t[s]
                    gh[j, d] = hv - jnp.where(hv >= HALFN, HALFN, 0)
                    gt[j, d] = tv - jnp.where(tv >= HALFN, HALFN, 0)
                    gr[j, d] = lax.shift_right_logical(idx_r[s], 1)

            copies = []
            for j in range(n_chunks):
                rsl = pl.ds(j * CHUNK, CHUNK)
                copies.append(pltpu.async_copy(
                    ent_hbm.at[gh.at[j]], rows_h.at[rsl], sem))
                copies.append(pltpu.async_copy(
                    relemb_hbm.at[gr.at[j]], rows_r.at[rsl], sem))
                copies.append(pltpu.async_copy(
                    ent_hbm.at[gt.at[j]], rows_t.at[rsl], sem))
            for c in copies:
                c.wait()

            def group(g, _):
                gsl = pl.ds(g * LANES, LANES)
                ovh = jnp.where(idx_h[gsl] >= HALFN, EMBED_DIM, 0)
                ovt = jnp.where(idx_t[gsl] >= HALFN, EMBED_DIM, 0)
                ovr = (idx_r[gsl] & 1) * EMBED_DIM
                for k in range(LANES):
                    i = g * LANES + k
                    oh, orr, ot = ovh[k], ovr[k], ovt[k]
                    for c in range(EMBED_DIM // LANES):
                        p = (rows_h[i, pl.ds(oh + c * LANES, LANES)]
                             * rows_r[i, pl.ds(orr + c * LANES, LANES)]
                             * rows_t[i, pl.ds(ot + c * LANES, LANES)])
                        rows_h[i, pl.ds(c * LANES, LANES)] = p
                return 0

            lax.fori_loop(0, ROUND // LANES, group, 0)

            out_cp = pltpu.async_copy(
                rows_h, out_hbm.at[pl.ds(rbase, ROUND)], out_sem)
            out_cp.wait()

    return sc_products


_sc_products = _make_sc_products()


def _tc_reduce_body(p_ref, out_ref):
    out_ref[...] = jnp.sum(p_ref[:, :EMBED_DIM], axis=1)


_TC_BLOCK = 2048


def _tc_reduce(products):
    return pl.pallas_call(
        _tc_reduce_body,
        out_shape=jax.ShapeDtypeStruct((BATCH,), jnp.float32),
        grid=(BATCH // _TC_BLOCK,),
        in_specs=[pl.BlockSpec((_TC_BLOCK, 128), lambda i: (i, 0))],
        out_specs=pl.BlockSpec((_TC_BLOCK,), lambda i: (i,)),
    )(products)


def kernel(head, relation, tail, entity_embeddings, relation_embeddings):
    ent2 = _to_gatherable(entity_embeddings.T)
    rel2 = relation_embeddings.reshape(-1, 128)
    products = _sc_products(head, relation, tail, ent2, rel2)
    return _tc_reduce(products)


# speedup vs baseline: 2.8091x; 1.0772x over previous
import functools

import jax
import jax.numpy as jnp
from jax import lax
from jax.experimental import pallas as pl
from jax.experimental.pallas import tpu as pltpu
from jax.experimental.pallas import tpu_sc as plsc

BATCH = 16384
EMBED_DIM = 64
LANES = 16
CHUNK = 128
ROUND = 256

NE = 1000000
_TW = 16384
HALFN = 31 * _TW


_DIMS = (((0,), (0,)), ((), ()))


def _transpose_body(a_ref, b_ref, out_ref):
    eye = jnp.eye(EMBED_DIM, dtype=jnp.bfloat16)
    out_ref[:, :EMBED_DIM] = lax.dot_general(
        a_ref[...].astype(jnp.bfloat16), eye, _DIMS,
        preferred_element_type=jnp.float32)
    out_ref[:, EMBED_DIM:] = lax.dot_general(
        b_ref[...].astype(jnp.bfloat16), eye, _DIMS,
        preferred_element_type=jnp.float32)


def _to_gatherable(ent_t):
    return pl.pallas_call(
        _transpose_body,
        out_shape=jax.ShapeDtypeStruct((HALFN, 128), jnp.float32),
        grid=(HALFN // _TW,),
        in_specs=[
            pl.BlockSpec((EMBED_DIM, _TW), lambda j: (0, j)),
            pl.BlockSpec(
                (EMBED_DIM, _TW),
                lambda j: (0, jnp.minimum(j + HALFN // _TW,
                                          (NE - 1) // _TW))),
        ],
        out_specs=pl.BlockSpec((_TW, 128), lambda j: (j, 0)),
    )(ent_t, ent_t)


def _make_sc_products():
    info = plsc.get_sparse_core_info()
    nc, ns = info.num_cores, info.num_subcores
    nw = nc * ns
    per_w = BATCH // nw
    n_rounds = per_w // ROUND
    n_chunks = ROUND // CHUNK

    mesh = plsc.VectorSubcoreMesh(core_axis_name="c", subcore_axis_name="s")

    @functools.partial(
        pl.kernel,
        mesh=mesh,
        out_type=jax.ShapeDtypeStruct((BATCH, 128), jnp.float32),
        scratch_types=[
            pltpu.VMEM((ROUND,), jnp.int32),
            pltpu.VMEM((ROUND,), jnp.int32),
            pltpu.VMEM((ROUND,), jnp.int32),
            pltpu.VMEM((n_chunks, CHUNK), jnp.int32),
            pltpu.VMEM((n_chunks, CHUNK), jnp.int32),
            pltpu.VMEM((n_chunks, CHUNK), jnp.int32),
            pltpu.VMEM((ROUND, 128), jnp.float32),
            pltpu.VMEM((ROUND, 128), jnp.float32),
            pltpu.VMEM((ROUND, 128), jnp.float32),
            pltpu.SemaphoreType.DMA,
            pltpu.SemaphoreType.DMA,
        ],
    )
    def sc_products(head_hbm, rel_hbm, tail_hbm, ent_hbm, relemb_hbm,
                    out_hbm, idx_h, idx_r, idx_t, gh, gr, gt,
                    rows_h, rows_r, rows_t, sem, out_sem):
        wid = lax.axis_index("s") * nc + lax.axis_index("c")
        base = wid * per_w

        for r in range(n_rounds):
            rbase = base + r * ROUND
            pltpu.sync_copy(head_hbm.at[pl.ds(rbase, ROUND)], idx_h)
            pltpu.sync_copy(rel_hbm.at[pl.ds(rbase, ROUND)], idx_r)
            pltpu.sync_copy(tail_hbm.at[pl.ds(rbase, ROUND)], idx_t)

            for j in range(n_chunks):
                for g in range(CHUNK // LANES):
                    s = pl.ds(j * CHUNK + g * LANES, LANES)
                    d = pl.ds(g * LANES, LANES)
                    hv = idx_h[s]
                    tv = idx_t[s]
                    gh[j, d] = hv - jnp.where(hv >= HALFN, HALFN, 0)
                    gt[j, d] = tv - jnp.where(tv >= HALFN, HALFN, 0)
                    gr[j, d] = lax.shift_right_logical(idx_r[s], 1)

            copies = []
            for j in range(n_chunks):
                rsl = pl.ds(j * CHUNK, CHUNK)
                copies.append(pltpu.async_copy(
                    ent_hbm.at[gh.at[j]], rows_h.at[rsl], sem))
                copies.append(pltpu.async_copy(
                    relemb_hbm.at[gr.at[j]], rows_r.at[rsl], sem))
                copies.append(pltpu.async_copy(
                    ent_hbm.at[gt.at[j]], rows_t.at[rsl], sem))
            for c in copies:
                c.wait()

            def group(g, _):
                gsl = pl.ds(g * LANES, LANES)
                ovh = jnp.where(idx_h[gsl] >= HALFN, EMBED_DIM, 0)
                ovt = jnp.where(idx_t[gsl] >= HALFN, EMBED_DIM, 0)
                ovr = (idx_r[gsl] & 1) * EMBED_DIM
                for k in range(LANES):
                    i = g * LANES + k
                    oh, orr, ot = ovh[k], ovr[k], ovt[k]
                    for c in range(EMBED_DIM // LANES):
                        p = (rows_h[i, pl.ds(oh + c * LANES, LANES)]
                             * rows_r[i, pl.ds(orr + c * LANES, LANES)]
                             * rows_t[i, pl.ds(ot + c * LANES, LANES)])
                        rows_h[i, pl.ds(c * LANES, LANES)] = p
                return 0

            lax.fori_loop(0, ROUND // LANES, group, 0)

            out_cp = pltpu.async_copy(
                rows_h, out_hbm.at[pl.ds(rbase, ROUND)], out_sem)
            out_cp.wait()

    return sc_products


_sc_products = _make_sc_products()


def _tc_reduce_body(p_ref, out_ref):
    out_ref[...] = jnp.sum(p_ref[:, :EMBED_DIM], axis=1)


_TC_BLOCK = 2048


def _tc_reduce(products):
    return pl.pallas_call(
        _tc_reduce_body,
        out_shape=jax.ShapeDtypeStruct((BATCH,), jnp.float32),
        grid=(BATCH // _TC_BLOCK,),
        in_specs=[pl.BlockSpec((_TC_BLOCK, 128), lambda i: (i, 0))],
        out_specs=pl.BlockSpec((_TC_BLOCK,), lambda i: (i,)),
    )(products)


def kernel(head, relation, tail, entity_embeddings, relation_embeddings):
    ent2 = _to_gatherable(entity_embeddings.T)
    rel2 = relation_embeddings.reshape(-1, 128)
    products = _sc_products(head, relation, tail, ent2, rel2)
    return _tc_reduce(products)
